# baseline (speedup 1.0000x reference)
import functools

import jax
import jax.numpy as jnp
from jax import lax
from jax.experimental import pallas as pl
from jax.experimental.pallas import tpu as pltpu
from jax.experimental.pallas import tpu_sc as plsc

NC = 2
NS = 16
NW = NC * NS


def _sc_gather3(pat_lat, meas_lat, time_lat, idx_pat, idx_meas, idx_t):
    B = idx_pat.shape[0]
    D = pat_lat.shape[1]
    b_per_w = B // NW
    out_t = jax.ShapeDtypeStruct((B, D), jnp.float32)
    mesh = plsc.VectorSubcoreMesh(core_axis_name="c", subcore_axis_name="s")

    @functools.partial(
        pl.kernel,
        mesh=mesh,
        out_type=(out_t, out_t, out_t),
        compiler_params=pltpu.CompilerParams(use_tc_tiling_on_sc=False),
        scratch_types=[
            pltpu.VMEM((b_per_w,), jnp.int32),
            pltpu.VMEM((b_per_w,), jnp.int32),
            pltpu.VMEM((b_per_w,), jnp.int32),
            pltpu.VMEM((b_per_w, 32), jnp.float32),
            pltpu.VMEM((b_per_w, 32), jnp.float32),
            pltpu.VMEM((b_per_w, 32), jnp.float32),
            pltpu.SemaphoreType.DMA,
            pltpu.SemaphoreType.DMA,
            pltpu.SemaphoreType.DMA,
        ],
    )
    def gather_kernel(pat_hbm, meas_hbm, tim_hbm, ip_hbm, im_hbm, it_hbm,
                      pat_out, meas_out, tim_out,
                      ipv, imv, itv, pv, mv, tv, sem_p, sem_m, sem_t):
        wid = lax.axis_index("s") * NC + lax.axis_index("c")
        base = wid * b_per_w
        pltpu.sync_copy(ip_hbm.at[pl.ds(base, b_per_w)], ipv)
        pltpu.sync_copy(im_hbm.at[pl.ds(base, b_per_w)], imv)
        pltpu.sync_copy(it_hbm.at[pl.ds(base, b_per_w)], itv)
        cp_p = pltpu.async_copy(pat_hbm.at[ipv], pv, sem_p)
        cp_m = pltpu.async_copy(meas_hbm.at[imv], mv, sem_m)
        cp_t = pltpu.async_copy(tim_hbm.at[itv], tv, sem_t)
        cp_p.wait()
        wb_p = pltpu.async_copy(pv, pat_out.at[pl.ds(base, b_per_w)], sem_p)
        cp_m.wait()
        wb_m = pltpu.async_copy(mv, meas_out.at[pl.ds(base, b_per_w)], sem_m)
        cp_t.wait()
        wb_t = pltpu.async_copy(tv, tim_out.at[pl.ds(base, b_per_w)], sem_t)
        wb_p.wait()
        wb_m.wait()
        wb_t.wait()

    return gather_kernel(pat_lat, meas_lat, time_lat, idx_pat, idx_meas, idx_t)


def _tc_combine_body(pat_ref, meas_ref, tim_ref, cu_ref, cw_ref,
                     bu_ref, bw_ref, s_ref, out_ref):
    u = jnp.dot(cu_ref[...], bu_ref[...],
                preferred_element_type=jnp.float32,
                precision=lax.Precision.HIGHEST)
    w = jnp.dot(cw_ref[...], bw_ref[...],
                preferred_element_type=jnp.float32,
                precision=lax.Precision.HIGHEST)
    prod = (pat_ref[...] + u) * meas_ref[...] * (tim_ref[...] + w)
    out_ref[...] = jnp.dot(prod, s_ref[...],
                           preferred_element_type=jnp.float32,
                           precision=lax.Precision.HIGHEST)


def kernel(idx_pat, idx_meas, idx_t, cov_u, cov_w, pat_lat, meas_lat,
           time_lat, beta_u, beta_w):
    B = idx_pat.shape[0]
    D = pat_lat.shape[1]
    NU = cov_u.shape[1]
    NWc = cov_w.shape[1]
    PACK = 128 // D
    R = B // PACK

    pat_g, meas_g, tim_g = _sc_gather3(
        pat_lat, meas_lat, time_lat,
        idx_pat.astype(jnp.int32), idx_meas.astype(jnp.int32),
        idx_t.astype(jnp.int32))

    pat4 = pat_g.reshape(R, PACK * D)
    meas4 = meas_g.reshape(R, PACK * D)
    tim4 = tim_g.reshape(R, PACK * D)
    cu4 = cov_u.reshape(R, PACK * NU)
    cw4 = cov_w.reshape(R, PACK * NWc)

    eye = jnp.eye(PACK, dtype=jnp.float32)
    bu_bd = jnp.kron(eye, beta_u)
    bw_bd = jnp.kron(eye, beta_w)
    s_mat = jnp.kron(eye, jnp.ones((D, 1), jnp.float32))

    out4 = pl.pallas_call(
        _tc_combine_body,
        out_shape=jax.ShapeDtypeStruct((R, PACK), jnp.float32),
    )(pat4, meas4, tim4, cu4, cw4, bu_bd, bw_bd, s_mat)
    return out4.reshape(B)


# speedup vs baseline: 1.4434x; 1.4434x over previous
import functools

import jax
import jax.numpy as jnp
from jax import lax
from jax.experimental import pallas as pl
from jax.experimental.pallas import tpu as pltpu
from jax.experimental.pallas import tpu_sc as plsc

NC = 2
NS = 16
NW = NC * NS


def _sc_gather3(pat_lat, meas_lat, time_lat, idx_pat, idx_meas, idx_t):
    B = idx_pat.shape[0]
    D = pat_lat.shape[1]
    b_per_w = B // NW
    out_t = jax.ShapeDtypeStruct((B, D), jnp.float32)
    mesh = plsc.VectorSubcoreMesh(core_axis_name="c", subcore_axis_name="s")

    @functools.partial(
        pl.kernel,
        mesh=mesh,
        out_type=(out_t, out_t, out_t),
        scratch_types=[
            pltpu.VMEM((b_per_w,), jnp.int32),
            pltpu.VMEM((b_per_w,), jnp.int32),
            pltpu.VMEM((b_per_w,), jnp.int32),
            pltpu.VMEM((b_per_w, 32), jnp.float32),
            pltpu.SemaphoreType.DMA,
            pltpu.SemaphoreType.DMA,
        ],
    )
    def gather_kernel(pat_hbm, meas_hbm, tim_hbm, ip_hbm, im_hbm, it_hbm,
                      pat_out, meas_out, tim_out,
                      ipv, imv, itv, buf, sem, sem_wb):
        wid = lax.axis_index("s") * NC + lax.axis_index("c")
        base = wid * b_per_w
        pltpu.sync_copy(ip_hbm.at[pl.ds(base, b_per_w)], ipv)
        pltpu.sync_copy(im_hbm.at[pl.ds(base, b_per_w)], imv)
        pltpu.sync_copy(it_hbm.at[pl.ds(base, b_per_w)], itv)

        def gather_one(tbl_hbm, idx_v, out_hbm):
            @pl.loop(0, b_per_w // 16)
            def _(g):
                k0 = g * 16
                iv = idx_v[pl.ds(k0, 16)]
                for j in range(16):
                    pltpu.async_copy(tbl_hbm.at[iv[j]], buf.at[k0 + j], sem)

            pltpu.make_async_copy(
                tbl_hbm.at[pl.ds(0, b_per_w)], buf, sem).wait()
            pltpu.async_copy(
                buf, out_hbm.at[pl.ds(base, b_per_w)], sem_wb).wait()

        gather_one(pat_hbm, ipv, pat_out)
        gather_one(meas_hbm, imv, meas_out)
        gather_one(tim_hbm, itv, tim_out)

    return gather_kernel(pat_lat, meas_lat, time_lat, idx_pat, idx_meas, idx_t)


def _tc_combine_body(pat_ref, meas_ref, tim_ref, cu_ref, cw_ref,
                     bu_ref, bw_ref, s_ref, out_ref):
    u = jnp.dot(cu_ref[...], bu_ref[...],
                preferred_element_type=jnp.float32,
                precision=lax.Precision.HIGHEST)
    w = jnp.dot(cw_ref[...], bw_ref[...],
                preferred_element_type=jnp.float32,
                precision=lax.Precision.HIGHEST)
    prod = (pat_ref[...] + u) * meas_ref[...] * (tim_ref[...] + w)
    out_ref[...] = jnp.dot(prod, s_ref[...],
                           preferred_element_type=jnp.float32,
                           precision=lax.Precision.HIGHEST)


def kernel(idx_pat, idx_meas, idx_t, cov_u, cov_w, pat_lat, meas_lat,
           time_lat, beta_u, beta_w):
    B = idx_pat.shape[0]
    D = pat_lat.shape[1]
    NU = cov_u.shape[1]
    NWc = cov_w.shape[1]
    PACK = 128 // D
    R = B // PACK

    pat_g, meas_g, tim_g = _sc_gather3(
        pat_lat, meas_lat, time_lat,
        idx_pat.astype(jnp.int32), idx_meas.astype(jnp.int32),
        idx_t.astype(jnp.int32))

    pat4 = pat_g.reshape(R, PACK * D)
    meas4 = meas_g.reshape(R, PACK * D)
    tim4 = tim_g.reshape(R, PACK * D)
    cu4 = cov_u.reshape(R, PACK * NU)
    cw4 = cov_w.reshape(R, PACK * NWc)

    eye = jnp.eye(PACK, dtype=jnp.float32)
    bu_bd = jnp.kron(eye, beta_u)
    bw_bd = jnp.kron(eye, beta_w)
    s_mat = jnp.kron(eye, jnp.ones((D, 1), jnp.float32))

    out4 = pl.pallas_call(
        _tc_combine_body,
        out_shape=jax.ShapeDtypeStruct((R, PACK), jnp.float32),
    )(pat4, meas4, tim4, cu4, cw4, bu_bd, bw_bd, s_mat)
    return out4.reshape(B)
